# baseline (speedup 1.0000x reference)
import functools

import jax
import jax.numpy as jnp
from jax import lax
from jax.experimental import pallas as pl
from jax.experimental.pallas import tpu as pltpu
from jax.experimental.pallas import tpu_sc as plsc

_CTX = 512
_NCLS = 4
_SEQ = 77
_PRE = 5
_SUF = _SEQ - _PRE - _NCLS


def _mlp_body(img_ref, w1_ref, b1_ref, w2_ref, b2_ref, out_ref):
    h = jnp.maximum(
        jnp.dot(img_ref[...], w1_ref[...], preferred_element_type=jnp.float32)
        + b1_ref[...],
        0.0,
    )
    out_ref[...] = (
        jnp.dot(h, w2_ref[...], preferred_element_type=jnp.float32) + b2_ref[...]
    )


def _run_mlp(img, W1, b1, W2, b2):
    B, F = img.shape
    H = W1.shape[1]
    O = W2.shape[1]
    BB = 256
    grid = (B // BB,)
    return pl.pallas_call(
        _mlp_body,
        grid=grid,
        in_specs=[
            pl.BlockSpec((BB, F), lambda i: (i, 0)),
            pl.BlockSpec((F, H), lambda i: (0, 0)),
            pl.BlockSpec((1, H), lambda i: (0, 0)),
            pl.BlockSpec((H, O), lambda i: (0, 0)),
            pl.BlockSpec((1, O), lambda i: (0, 0)),
        ],
        out_specs=pl.BlockSpec((BB, O), lambda i: (i, 0)),
        out_shape=jax.ShapeDtypeStruct((B, O), jnp.float32),
    )(img, W1, b1.reshape(1, H), W2, b2.reshape(1, O))


def _make_assemble(B):
    info = plsc.get_sparse_core_info()
    nc, ns = info.num_cores, info.num_subcores
    nw = nc * ns
    bpw = B // nw
    mesh = plsc.VectorSubcoreMesh(core_axis_name="c", subcore_axis_name="s")

    @functools.partial(
        pl.kernel,
        out_type=jax.ShapeDtypeStruct((B, _SEQ, _CTX), jnp.float32),
        mesh=mesh,
        compiler_params=pltpu.CompilerParams(use_tc_tiling_on_sc=False),
        scratch_types=[
            pltpu.VMEM((_PRE, _CTX), jnp.float32),
            pltpu.VMEM((_SUF, _CTX), jnp.float32),
            pltpu.VMEM((2, _NCLS, _CTX), jnp.float32),
            pltpu.SemaphoreType.DMA,
            pltpu.SemaphoreType.DMA,
            pltpu.SemaphoreType.DMA,
            pltpu.SemaphoreType.DMA,
        ],
    )
    def assemble(pre_hbm, suf_hbm, cls_hbm, out_hbm, pre_v, suf_v, cls_v,
                 sem_in, sem_out, sem_c0, sem_c1):
        wid = lax.axis_index("s") * nc + lax.axis_index("c")
        base = wid * bpw
        pltpu.sync_copy(pre_hbm, pre_v)
        pltpu.sync_copy(suf_hbm, suf_v)
        sem_c = (sem_c0, sem_c1)
        tail = []
        slot_cls_dma = [None, None]
        for i in range(bpw):
            b = base + i
            s = i % 2
            cp_in = pltpu.make_async_copy(cls_hbm.at[b], cls_v.at[s], sem_in)
            cp_in.start()
            if slot_cls_dma[s] is not None:
                slot_cls_dma[s].wait()
            cp_in.wait()
            cp_p = pltpu.make_async_copy(pre_v, out_hbm.at[b, pl.ds(0, _PRE)],
                                         sem_out)
            cp_c = pltpu.make_async_copy(cls_v.at[s],
                                         out_hbm.at[b, pl.ds(_PRE, _NCLS)],
                                         sem_c[s])
            cp_s = pltpu.make_async_copy(suf_v,
                                         out_hbm.at[b, pl.ds(_PRE + _NCLS, _SUF)],
                                         sem_out)
            cp_p.start()
            cp_c.start()
            cp_s.start()
            tail += [cp_p, cp_s]
            slot_cls_dma[s] = cp_c
        for cp in tail:
            cp.wait()
        for cp in slot_cls_dma:
            if cp is not None:
                cp.wait()

    return assemble


def kernel(img, W1, b1, W2, b2, token_prefix, token_suffix):
    B = img.shape[0]
    cls = _run_mlp(img, W1, b1, W2, b2).reshape(B, _NCLS, _CTX)
    pre = token_prefix.reshape(_PRE, _CTX)
    suf = token_suffix.reshape(_SUF, _CTX)
    return _make_assemble(B)(pre, suf, cls)


# speedup vs baseline: 1.0292x; 1.0292x over previous
import functools

import jax
import jax.numpy as jnp
from jax import lax
from jax.experimental import pallas as pl
from jax.experimental.pallas import tpu as pltpu
from jax.experimental.pallas import tpu_sc as plsc

_CTX = 512
_NCLS = 4
_SEQ = 77
_PRE = 5
_SUF = _SEQ - _PRE - _NCLS


def _mlp_body(img_ref, w1_ref, b1_ref, w2_ref, b2_ref, out_ref):
    h = jnp.maximum(
        jnp.dot(img_ref[...], w1_ref[...], preferred_element_type=jnp.float32)
        + b1_ref[...],
        0.0,
    )
    out_ref[...] = (
        jnp.dot(h, w2_ref[...], preferred_element_type=jnp.float32) + b2_ref[...]
    )


def _run_mlp(img, W1, b1, W2, b2):
    B, F = img.shape
    H = W1.shape[1]
    O = W2.shape[1]
    BB = 256
    grid = (B // BB,)
    return pl.pallas_call(
        _mlp_body,
        grid=grid,
        in_specs=[
            pl.BlockSpec((BB, F), lambda i: (i, 0)),
            pl.BlockSpec((F, H), lambda i: (0, 0)),
            pl.BlockSpec((1, H), lambda i: (0, 0)),
            pl.BlockSpec((H, O), lambda i: (0, 0)),
            pl.BlockSpec((1, O), lambda i: (0, 0)),
        ],
        out_specs=pl.BlockSpec((BB, O), lambda i: (i, 0)),
        out_shape=jax.ShapeDtypeStruct((B, O), jnp.float32),
    )(img, W1, b1.reshape(1, H), W2, b2.reshape(1, O))


def _make_assemble(B):
    info = plsc.get_sparse_core_info()
    nc, ns = info.num_cores, info.num_subcores
    nw = nc * ns
    bpw = B // nw
    mesh = plsc.VectorSubcoreMesh(core_axis_name="c", subcore_axis_name="s")

    @functools.partial(
        pl.kernel,
        out_type=jax.ShapeDtypeStruct((B, _SEQ, _CTX), jnp.float32),
        mesh=mesh,
        compiler_params=pltpu.CompilerParams(use_tc_tiling_on_sc=False),
        scratch_types=[
            pltpu.VMEM((_PRE, _CTX), jnp.float32),
            pltpu.VMEM((_SUF, _CTX), jnp.float32),
            pltpu.VMEM((bpw, _NCLS, _CTX), jnp.float32),
            pltpu.SemaphoreType.DMA,
            pltpu.SemaphoreType.DMA,
        ],
    )
    def assemble(pre_hbm, suf_hbm, cls_hbm, out_hbm, pre_v, suf_v, cls_v,
                 sem_in, sem_out):
        wid = lax.axis_index("s") * nc + lax.axis_index("c")
        base = wid * bpw
        cp_cls = pltpu.make_async_copy(cls_hbm.at[pl.ds(base, bpw)], cls_v,
                                       sem_in)
        cp_cls.start()
        pltpu.sync_copy(pre_hbm, pre_v)
        pltpu.sync_copy(suf_hbm, suf_v)
        cp_cls.wait()
        tail = []
        for i in range(bpw):
            b = base + i
            cp_p = pltpu.make_async_copy(pre_v, out_hbm.at[b, pl.ds(0, _PRE)],
                                         sem_out)
            cp_c = pltpu.make_async_copy(cls_v.at[i],
                                         out_hbm.at[b, pl.ds(_PRE, _NCLS)],
                                         sem_out)
            cp_s = pltpu.make_async_copy(suf_v,
                                         out_hbm.at[b, pl.ds(_PRE + _NCLS, _SUF)],
                                         sem_out)
            cp_p.start()
            cp_c.start()
            cp_s.start()
            tail += [cp_p, cp_c, cp_s]
        for cp in tail:
            cp.wait()

    return assemble


def kernel(img, W1, b1, W2, b2, token_prefix, token_suffix):
    B = img.shape[0]
    cls = _run_mlp(img, W1, b1, W2, b2).reshape(B, _NCLS, _CTX)
    pre = token_prefix.reshape(_PRE, _CTX)
    suf = token_suffix.reshape(_SUF, _CTX)
    return _make_assemble(B)(pre, suf, cls)


# speedup vs baseline: 1.9936x; 1.9371x over previous
import functools

import jax
import jax.numpy as jnp
from jax import lax
from jax.experimental import pallas as pl
from jax.experimental.pallas import tpu as pltpu
from jax.experimental.pallas import tpu_sc as plsc

_CTX = 512
_NCLS = 4
_SEQ = 77
_PRE = 5
_SUF = _SEQ - _PRE - _NCLS
_HEAD = 16
_REST = _SEQ - _HEAD
_SUF_HEAD = _HEAD - _PRE - _NCLS


def _make_fill_rest(B):
    info = plsc.get_sparse_core_info()
    nc, ns = info.num_cores, info.num_subcores
    nw = nc * ns
    bpw = B // nw
    mesh = plsc.VectorSubcoreMesh(core_axis_name="c", subcore_axis_name="s")

    @functools.partial(
        pl.kernel,
        out_type=jax.ShapeDtypeStruct((B, _SEQ, _CTX), jnp.float32),
        mesh=mesh,
        scratch_types=[
            pltpu.VMEM((_REST, _CTX), jnp.float32),
            pltpu.SemaphoreType.DMA,
        ],
    )
    def fill_rest(rest_hbm, out_hbm, rest_v, sem_out):
        wid = lax.axis_index("s") * nc + lax.axis_index("c")
        base = wid * bpw
        pltpu.sync_copy(rest_hbm, rest_v)
        tail = []
        for i in range(bpw):
            cp = pltpu.make_async_copy(
                rest_v, out_hbm.at[base + i, pl.ds(_HEAD, _REST)], sem_out)
            cp.start()
            tail.append(cp)
        for cp in tail:
            cp.wait()

    return fill_rest


def _head_body(rest_ref, img_ref, w1_ref, b1_ref, w2_ref, b2_ref, pre_ref,
               suf7_ref, out_ref):
    del rest_ref
    bb = img_ref.shape[0]
    h = jnp.maximum(
        jnp.dot(img_ref[...], w1_ref[...], preferred_element_type=jnp.float32)
        + b1_ref[...],
        0.0,
    )
    out_ref[:, 0:_PRE, :] = jnp.broadcast_to(pre_ref[...][None],
                                             (bb, _PRE, _CTX))
    for j in range(_NCLS):
        out_ref[:, _PRE + j, :] = (
            jnp.dot(h, w2_ref[:, j * _CTX:(j + 1) * _CTX],
                    preferred_element_type=jnp.float32)
            + b2_ref[:, j * _CTX:(j + 1) * _CTX]
        )
    out_ref[:, _PRE + _NCLS:_HEAD, :] = jnp.broadcast_to(
        suf7_ref[...][None], (bb, _SUF_HEAD, _CTX))


def _fill_head(rest_filled, img, W1, b1, W2, b2, pre, suf7):
    B, F = img.shape
    H = W1.shape[1]
    O = W2.shape[1]
    BB = 128
    grid = (B // BB,)
    return pl.pallas_call(
        _head_body,
        grid=grid,
        in_specs=[
            pl.BlockSpec(memory_space=pltpu.MemorySpace.HBM),
            pl.BlockSpec((BB, F), lambda i: (i, 0)),
            pl.BlockSpec((F, H), lambda i: (0, 0)),
            pl.BlockSpec((1, H), lambda i: (0, 0)),
            pl.BlockSpec((H, O), lambda i: (0, 0)),
            pl.BlockSpec((1, O), lambda i: (0, 0)),
            pl.BlockSpec((_PRE, _CTX), lambda i: (0, 0)),
            pl.BlockSpec((_SUF_HEAD, _CTX), lambda i: (0, 0)),
        ],
        out_specs=pl.BlockSpec((BB, _HEAD, _CTX), lambda i: (i, 0, 0)),
        out_shape=jax.ShapeDtypeStruct((B, _SEQ, _CTX), jnp.float32),
        input_output_aliases={0: 0},
    )(rest_filled, img, W1, b1.reshape(1, H), W2, b2.reshape(1, O), pre, suf7)


def kernel(img, W1, b1, W2, b2, token_prefix, token_suffix):
    B = img.shape[0]
    pre = token_prefix.reshape(_PRE, _CTX)
    suf = token_suffix.reshape(_SUF, _CTX)
    rest_filled = _make_fill_rest(B)(suf[_SUF_HEAD:])
    return _fill_head(rest_filled, img, W1, b1, W2, b2, pre, suf[:_SUF_HEAD])


# speedup vs baseline: 4.2657x; 2.1397x over previous
import functools

import jax
import jax.numpy as jnp
from jax import lax
from jax.experimental import pallas as pl
from jax.experimental.pallas import tpu as pltpu
from jax.experimental.pallas import tpu_sc as plsc

_CTX = 512
_NCLS = 4
_SEQ = 77
_PRE = 5
_SUF = _SEQ - _PRE - _NCLS
_NCONST = _PRE + _SUF
_REP = 32
_NSLOT = 4


def _make_fill_const(B):
    info = plsc.get_sparse_core_info()
    nc, ns = info.num_cores, info.num_subcores
    nw = nc * ns
    nchunk = B // _REP
    upw = _NCONST * nchunk // nw
    assert _NCONST * nchunk == upw * nw
    mesh = plsc.VectorSubcoreMesh(core_axis_name="c", subcore_axis_name="s")

    @functools.partial(
        pl.kernel,
        out_type=jax.ShapeDtypeStruct((_SEQ, B, _CTX), jnp.float32),
        mesh=mesh,
        scratch_types=[
            pltpu.VMEM((_NSLOT, _REP, _CTX), jnp.float32),
            pltpu.SemaphoreType.DMA,
            pltpu.SemaphoreType.DMA,
        ],
    )
    def fill_const(rep_hbm, out_hbm, buf_v, sem_in, sem_out):
        wid = lax.axis_index("s") * nc + lax.axis_index("c")
        u0 = wid * upw
        row0 = u0 // nchunk
        loads = []
        for i in range(_NSLOT):
            k = jnp.minimum(row0 + i, _NCONST - 1)
            cp = pltpu.make_async_copy(rep_hbm.at[k], buf_v.at[i], sem_in)
            cp.start()
            loads.append(cp)
        for cp in loads:
            cp.wait()
        tail = []
        for j in range(upw):
            u = u0 + j
            k = u // nchunk
            c = u % nchunk
            slot = k - row0
            r = jnp.where(k < _PRE, k, k + _NCLS)
            off = pl.multiple_of(c * _REP, _REP)
            cp = pltpu.make_async_copy(
                buf_v.at[slot], out_hbm.at[r, pl.ds(off, _REP)], sem_out)
            cp.start()
            tail.append(cp)
        for cp in tail:
            cp.wait()

    return fill_const


def _cls_body(const_ref, img_ref, w1_ref, b1_ref, w2_ref, b2_ref, out_ref):
    del const_ref
    h = jnp.maximum(
        jnp.dot(img_ref[...], w1_ref[...], preferred_element_type=jnp.float32)
        + b1_ref[...],
        0.0,
    )
    out_ref[0] = (
        jnp.dot(h, w2_ref[...], preferred_element_type=jnp.float32)
        + b2_ref[0]
    )


def _fill_cls(const_filled, img, W1, b1, W2, b2):
    B, F = img.shape
    H = W1.shape[1]
    BB = 256
    grid = (B // BB, _NCLS)
    return pl.pallas_call(
        _cls_body,
        grid=grid,
        in_specs=[
            pl.BlockSpec(memory_space=pltpu.MemorySpace.HBM),
            pl.BlockSpec((BB, F), lambda i, j: (i, 0)),
            pl.BlockSpec((F, H), lambda i, j: (0, 0)),
            pl.BlockSpec((1, H), lambda i, j: (0, 0)),
            pl.BlockSpec((H, _CTX), lambda i, j: (0, j)),
            pl.BlockSpec((1, 1, _CTX), lambda i, j: (j, 0, 0)),
        ],
        out_specs=pl.BlockSpec((1, BB, _CTX), lambda i, j: (_PRE + j, i, 0)),
        out_shape=jax.ShapeDtypeStruct((_SEQ, B, _CTX), jnp.float32),
        input_output_aliases={0: 0},
    )(const_filled, img, W1, b1.reshape(1, H), W2,
      b2.reshape(_NCLS, 1, _CTX))


def kernel(img, W1, b1, W2, b2, token_prefix, token_suffix):
    B = img.shape[0]
    pre = token_prefix.reshape(_PRE, _CTX)
    suf = token_suffix.reshape(_SUF, _CTX)
    const_rows = jnp.concatenate([pre, suf], axis=0)
    rep = jnp.broadcast_to(const_rows[:, None, :], (_NCONST, _REP, _CTX))
    const_filled = _make_fill_const(B)(rep)
    out_t = _fill_cls(const_filled, img, W1, b1, W2, b2)
    return jnp.transpose(out_t, (1, 0, 2))


# speedup vs baseline: 4.5864x; 1.0752x over previous
import functools

import jax
import jax.numpy as jnp
from jax import lax
from jax.experimental import pallas as pl
from jax.experimental.pallas import tpu as pltpu
from jax.experimental.pallas import tpu_sc as plsc

_CTX = 512
_NCLS = 4
_SEQ = 77
_PRE = 5
_SUF = _SEQ - _PRE - _NCLS
_NCONST = _PRE + _SUF
_REP = 32
_NSLOT = 4


def _make_fill_const(B):
    info = plsc.get_sparse_core_info()
    nc, ns = info.num_cores, info.num_subcores
    nw = nc * ns
    nchunk = B // _REP
    upw = _NCONST * nchunk // nw
    assert _NCONST * nchunk == upw * nw
    mesh = plsc.VectorSubcoreMesh(core_axis_name="c", subcore_axis_name="s")

    @functools.partial(
        pl.kernel,
        out_type=jax.ShapeDtypeStruct((_SEQ, B, _CTX), jnp.float32),
        mesh=mesh,
        scratch_types=[
            pltpu.VMEM((_NSLOT, _REP, _CTX), jnp.float32),
            pltpu.SemaphoreType.DMA,
            pltpu.SemaphoreType.DMA,
        ],
    )
    def fill_const(rep_hbm, out_hbm, buf_v, sem_in, sem_out):
        wid = lax.axis_index("s") * nc + lax.axis_index("c")
        u0 = wid * upw
        row0 = u0 // nchunk
        loads = []
        for i in range(_NSLOT):
            k = jnp.minimum(row0 + i, _NCONST - 1)
            cp = pltpu.make_async_copy(rep_hbm.at[k], buf_v.at[i], sem_in)
            cp.start()
            loads.append(cp)
        for cp in loads:
            cp.wait()
        tail = []
        for j in range(upw):
            u = u0 + j
            k = u // nchunk
            c = u % nchunk
            slot = k - row0
            r = jnp.where(k < _PRE, k, k + _NCLS)
            off = pl.multiple_of(c * _REP, _REP)
            cp = pltpu.make_async_copy(
                buf_v.at[slot], out_hbm.at[r, pl.ds(off, _REP)], sem_out)
            cp.start()
            tail.append(cp)
        for cp in tail:
            cp.wait()

    return fill_const


def _cls_body(const_ref, img_ref, w1_ref, b1_ref, w2_ref, b2_ref, out_ref,
              h_ref):
    del const_ref
    j = pl.program_id(1)

    @pl.when(j == 0)
    def _():
        h_ref[...] = jnp.maximum(
            jnp.dot(img_ref[...], w1_ref[...],
                    preferred_element_type=jnp.float32) + b1_ref[...],
            0.0,
        )

    out_ref[0] = (
        jnp.dot(h_ref[...], w2_ref[...], preferred_element_type=jnp.float32)
        + b2_ref[0]
    )


def _fill_cls(const_filled, img, W1, b1, W2, b2):
    B, F = img.shape
    H = W1.shape[1]
    BB = 512
    grid = (B // BB, _NCLS)
    return pl.pallas_call(
        _cls_body,
        grid=grid,
        in_specs=[
            pl.BlockSpec(memory_space=pltpu.MemorySpace.HBM),
            pl.BlockSpec((BB, F), lambda i, j: (i, 0)),
            pl.BlockSpec((F, H), lambda i, j: (0, 0)),
            pl.BlockSpec((1, H), lambda i, j: (0, 0)),
            pl.BlockSpec((H, _CTX), lambda i, j: (0, j)),
            pl.BlockSpec((1, 1, _CTX), lambda i, j: (j, 0, 0)),
        ],
        out_specs=pl.BlockSpec((1, BB, _CTX), lambda i, j: (_PRE + j, i, 0)),
        out_shape=jax.ShapeDtypeStruct((_SEQ, B, _CTX), jnp.float32),
        input_output_aliases={0: 0},
        scratch_shapes=[pltpu.VMEM((BB, H), jnp.float32)],
    )(const_filled, img, W1, b1.reshape(1, H), W2,
      b2.reshape(_NCLS, 1, _CTX))


def kernel(img, W1, b1, W2, b2, token_prefix, token_suffix):
    B = img.shape[0]
    pre = token_prefix.reshape(_PRE, _CTX)
    suf = token_suffix.reshape(_SUF, _CTX)
    const_rows = jnp.concatenate([pre, suf], axis=0)
    rep = jnp.broadcast_to(const_rows[:, None, :], (_NCONST, _REP, _CTX))
    const_filled = _make_fill_const(B)(rep)
    out_t = _fill_cls(const_filled, img, W1, b1, W2, b2)
    return jnp.transpose(out_t, (1, 0, 2))
